# baseline (speedup 1.0000x reference)
import jax
import jax.numpy as jnp
from jax import lax
from jax.experimental import pallas as pl
from jax.experimental.pallas import tpu as pltpu
from jax.experimental.pallas import tpu_sc as plsc

_B = 16384
_D = 32
_NC = 2
_NS = 16
_NW = _NC * _NS
_BPW = _B // _NW
_CH = 128
_NCH = _BPW // _CH

_BM = 1024
_NB = _B // _BM


def _sc_gather_body(w1, w2, idx1, idx2, emb, idx1_v, idx2_v, rows1_v, rows2_v, sem):
    wid = lax.axis_index("s") * _NC + lax.axis_index("c")
    base = wid * _BPW
    pltpu.sync_copy(idx1.at[wid], idx1_v)
    pltpu.sync_copy(idx2.at[wid], idx2_v)
    copies = []
    for j in range(_NCH):
        copies.append(
            pltpu.async_copy(w1.at[idx1_v.at[j]], rows1_v.at[pl.ds(j * _CH, _CH)], sem))
        copies.append(
            pltpu.async_copy(w2.at[idx2_v.at[j]], rows2_v.at[pl.ds(j * _CH, _CH)], sem))
    for c in copies:
        c.wait()
    pltpu.sync_copy(rows1_v, emb.at[0, pl.ds(base, _BPW)])
    pltpu.sync_copy(rows2_v, emb.at[1, pl.ds(base, _BPW)])


def _tc_mlp_body(e1_ref, e2_ref, c1_ref, c2_ref, ck_ref, lk_ref, cm_ref,
                 wc, bc, gc, bec, wl, bl, gl, bel, wm, bm, gm, bem,
                 st_ref, out_ref):
    m1 = (c1_ref[0, 0, :] != 0).astype(jnp.float32)[:, None]
    m2 = (c2_ref[0, 0, :] != 0).astype(jnp.float32)[:, None]
    out_ref[:, 0:_D] = e1_ref[0] * m1
    out_ref[:, _D:2 * _D] = e2_ref[0] * m2

    def mlp(x, w, b, g, be, m, s):
        z = (jnp.log1p(x) - m) / s
        h = jnp.maximum(z[:, None] * w[0] + b[0], 0.0)
        mu = h.mean(-1, keepdims=True)
        var = ((h - mu) ** 2).mean(-1, keepdims=True)
        return (h - mu) * lax.rsqrt(var + 1e-5) * g[0] + be[0]

    out_ref[:, 2 * _D:3 * _D] = mlp(ck_ref[0, 0, :], wc, bc, gc, bec,
                                    st_ref[0], st_ref[1])
    out_ref[:, 3 * _D:4 * _D] = mlp(lk_ref[0, 0, :], wl, bl, gl, bel,
                                    st_ref[2], st_ref[3])
    out_ref[:, 4 * _D:5 * _D] = mlp(cm_ref[0, 0, :], wm, bm, gm, bem,
                                    st_ref[4], st_ref[5])


def kernel(category_first, category_second, click_count, like_count, comment_count,
           W_cat1, W_cat2,
           w_click, b_click, g_click, be_click,
           w_like, b_like, g_like, be_like,
           w_comment, b_comment, g_comment, be_comment,
           m_click, s_click, m_like, s_like, m_comment, s_comment):
    idx1 = category_first.astype(jnp.int32)
    idx2 = category_second.astype(jnp.int32)

    sc_gather = pl.kernel(
        _sc_gather_body,
        out_type=jax.ShapeDtypeStruct((2, _B, _D), jnp.float32),
        mesh=plsc.VectorSubcoreMesh(core_axis_name="c", subcore_axis_name="s"),
        scratch_types=[
            pltpu.VMEM((_NCH, _CH), jnp.int32),
            pltpu.VMEM((_NCH, _CH), jnp.int32),
            pltpu.VMEM((_BPW, _D), jnp.float32),
            pltpu.VMEM((_BPW, _D), jnp.float32),
            pltpu.SemaphoreType.DMA,
        ],
        compiler_params=pltpu.CompilerParams(use_tc_tiling_on_sc=False),
    )
    emb = sc_gather(W_cat1, W_cat2,
                    idx1.reshape(_NW, _NCH, _CH), idx2.reshape(_NW, _NCH, _CH))

    stats = jnp.stack([m_click, s_click, m_like, s_like, m_comment, s_comment])

    row_spec = pl.BlockSpec((1, 1, _BM), lambda i: (i, 0, 0))
    par_spec = pl.BlockSpec((1, _D), lambda i: (0, 0))
    out = pl.pallas_call(
        _tc_mlp_body,
        grid=(_NB,),
        in_specs=[
            pl.BlockSpec((1, _BM, _D), lambda i: (0, i, 0)),
            pl.BlockSpec((1, _BM, _D), lambda i: (1, i, 0)),
            row_spec, row_spec, row_spec, row_spec, row_spec,
            par_spec, par_spec, par_spec, par_spec,
            par_spec, par_spec, par_spec, par_spec,
            par_spec, par_spec, par_spec, par_spec,
            pl.BlockSpec(memory_space=pltpu.SMEM),
        ],
        out_specs=pl.BlockSpec((_BM, 5 * _D), lambda i: (i, 0)),
        out_shape=jax.ShapeDtypeStruct((_B, 5 * _D), jnp.float32),
    )(emb, emb,
      idx1.reshape(_NB, 1, _BM), idx2.reshape(_NB, 1, _BM),
      click_count.reshape(_NB, 1, _BM), like_count.reshape(_NB, 1, _BM),
      comment_count.reshape(_NB, 1, _BM),
      w_click, b_click.reshape(1, _D), g_click.reshape(1, _D), be_click.reshape(1, _D),
      w_like, b_like.reshape(1, _D), g_like.reshape(1, _D), be_like.reshape(1, _D),
      w_comment, b_comment.reshape(1, _D), g_comment.reshape(1, _D), be_comment.reshape(1, _D),
      stats)
    return out
